# Initial kernel scaffold; baseline (speedup 1.0000x reference)
import jax
import jax.numpy as jnp
from jax.experimental import pallas as pl


def kernel(batch, emb_table):
    raise NotImplementedError("write your pallas kernel here")


# speedup vs baseline: 6.9363x; 6.9363x over previous
import functools

import jax
import jax.numpy as jnp
from jax import lax
from jax.experimental import pallas as pl
from jax.experimental.pallas import tpu as pltpu
from jax.experimental.pallas import tpu_sc as plsc

N_ROWS = 100001
D = 128
B_TOTAL = 1024 * 200
NUM_WORKERS = 32
B_PER_W = B_TOTAL // NUM_WORKERS
CHUNK = 400
N_CHUNKS = B_PER_W // CHUNK

_mesh = plsc.VectorSubcoreMesh(core_axis_name="c", subcore_axis_name="s")


@functools.partial(
    pl.kernel,
    mesh=_mesh,
    out_type=jax.ShapeDtypeStruct((B_TOTAL, D), jnp.float32),
    scratch_types=[
        pltpu.VMEM((CHUNK,), jnp.int32),
        pltpu.VMEM((CHUNK, D), jnp.float32),
        pltpu.SemaphoreType.DMA,
    ],
)
def _gather_kernel(idx_hbm, table_hbm, out_hbm, idx_v, rows_v, sem):
    wid = lax.axis_index("s") * 2 + lax.axis_index("c")
    base = wid * B_PER_W

    def body(g, carry):
        off = base + g * CHUNK
        pltpu.sync_copy(idx_hbm.at[pl.ds(off, CHUNK)], idx_v)
        pltpu.async_copy(table_hbm.at[idx_v], rows_v, sem).wait()
        pltpu.sync_copy(rows_v, out_hbm.at[pl.ds(off, CHUNK)])
        return carry

    lax.fori_loop(0, N_CHUNKS, body, 0)


def kernel(batch, emb_table):
    idx = batch.reshape(-1)
    out = _gather_kernel(idx, emb_table)
    return out.reshape(batch.shape[0], batch.shape[1], D)


# speedup vs baseline: 8.0362x; 1.1586x over previous
import functools

import jax
import jax.numpy as jnp
from jax import lax
from jax.experimental import pallas as pl
from jax.experimental.pallas import tpu as pltpu
from jax.experimental.pallas import tpu_sc as plsc

N_ROWS = 100001
D = 128
B_TOTAL = 1024 * 200
NUM_WORKERS = 32
B_PER_W = B_TOTAL // NUM_WORKERS
CHUNK = 400
N_CHUNKS = B_PER_W // CHUNK

_mesh = plsc.VectorSubcoreMesh(core_axis_name="c", subcore_axis_name="s")


@functools.partial(
    pl.kernel,
    mesh=_mesh,
    out_type=jax.ShapeDtypeStruct((B_TOTAL, D), jnp.float32),
    scratch_types=[
        pltpu.VMEM((B_PER_W,), jnp.int32),
        pltpu.VMEM((CHUNK, D), jnp.float32),
        pltpu.VMEM((CHUNK, D), jnp.float32),
        pltpu.SemaphoreType.DMA,
        pltpu.SemaphoreType.DMA,
        pltpu.SemaphoreType.DMA,
        pltpu.SemaphoreType.DMA,
    ],
)
def _gather_kernel(idx_hbm, table_hbm, out_hbm, idx_all, rows0, rows1,
                   gs0, gs1, ss0, ss1):
    wid = lax.axis_index("s") * 2 + lax.axis_index("c")
    base = wid * B_PER_W
    rows = (rows0, rows1)
    gsem = (gs0, gs1)
    ssem = (ss0, ss1)

    pltpu.sync_copy(idx_hbm.at[pl.ds(base, B_PER_W)], idx_all)

    def gather_start(g, b):
        pltpu.async_copy(
            table_hbm.at[idx_all.at[pl.ds(g * CHUNK, CHUNK)]], rows[b], gsem[b])

    def gather_wait(g, b):
        pltpu.make_async_copy(
            table_hbm.at[idx_all.at[pl.ds(g * CHUNK, CHUNK)]], rows[b], gsem[b]).wait()

    def store_start(g, b):
        pltpu.async_copy(rows[b], out_hbm.at[pl.ds(base + g * CHUNK, CHUNK)], ssem[b])

    def store_wait(g, b):
        pltpu.make_async_copy(
            rows[b], out_hbm.at[pl.ds(base + g * CHUNK, CHUNK)], ssem[b]).wait()

    gather_start(0, 0)
    gather_start(1, 1)

    def outer(go, carry):
        for b in range(2):
            g = 2 * go + b
            gather_wait(g, b)
            store_start(g, b)
            store_wait(g, b)
            gather_start(g + 2, b)
        return carry

    lax.fori_loop(0, N_CHUNKS // 2 - 1, outer, 0)

    for b in range(2):
        g = N_CHUNKS - 2 + b
        gather_wait(g, b)
        store_start(g, b)
    for b in range(2):
        g = N_CHUNKS - 2 + b
        store_wait(g, b)


def kernel(batch, emb_table):
    idx = batch.reshape(-1)
    out = _gather_kernel(idx, emb_table)
    return out.reshape(batch.shape[0], batch.shape[1], D)
